# Initial kernel scaffold; baseline (speedup 1.0000x reference)
import jax
import jax.numpy as jnp
from jax.experimental import pallas as pl


def kernel(idx, emb1, emb2):
    raise NotImplementedError("write your pallas kernel here")


# speedup vs baseline: 1.5334x; 1.5334x over previous
import functools

import jax
import jax.numpy as jnp
from jax import lax
from jax.experimental import pallas as pl
from jax.experimental.pallas import tpu as pltpu
from jax.experimental.pallas import tpu_sc as plsc

NNODES = 100000
DIM = 128
BATCH = 16384

_info = plsc.get_sparse_core_info()
_NC, _NS = _info.num_cores, _info.num_subcores
_NW = _NC * _NS
_B_PER_W = BATCH // _NW
_CHUNK = 128
_NCHUNK = _B_PER_W // _CHUNK

_mesh = plsc.VectorSubcoreMesh(core_axis_name="c", subcore_axis_name="s")


@functools.partial(
    pl.kernel,
    mesh=_mesh,
    out_type=(jax.ShapeDtypeStruct((BATCH, DIM), jnp.float32),
              jax.ShapeDtypeStruct((BATCH, DIM), jnp.float32)),
    scratch_types=[
        pltpu.VMEM((_NCHUNK, _CHUNK), jnp.int32),
        pltpu.VMEM((_CHUNK, DIM), jnp.float32),
        pltpu.VMEM((_CHUNK, DIM), jnp.float32),
        pltpu.VMEM((_CHUNK, DIM), jnp.float32),
        pltpu.VMEM((_CHUNK, DIM), jnp.float32),
        pltpu.SemaphoreType.DMA,
        pltpu.SemaphoreType.DMA,
        pltpu.SemaphoreType.DMA,
        pltpu.SemaphoreType.DMA,
    ],
)
def _lookup(idx_hbm, emb1_hbm, emb2_hbm, out1_hbm, out2_hbm,
            idx_v, r1a, r1b, r2a, r2b, s1a, s1b, s2a, s2b):
    wid = lax.axis_index("s") * _NC + lax.axis_index("c")
    base = wid * _B_PER_W
    pltpu.sync_copy(idx_hbm.at[wid], idx_v)
    bufs1, bufs2 = (r1a, r1b), (r2a, r2b)
    sems1, sems2 = (s1a, s1b), (s2a, s2b)
    pending = None
    for c in range(_NCHUNK):
        b = c % 2
        cp1 = pltpu.async_copy(emb1_hbm.at[idx_v.at[c]], bufs1[b], sems1[b])
        cp2 = pltpu.async_copy(emb2_hbm.at[idx_v.at[c]], bufs2[b], sems2[b])
        if pending is not None:
            pc, pb, pcp1, pcp2 = pending
            off = base + pc * _CHUNK
            pcp1.wait()
            pltpu.sync_copy(bufs1[pb], out1_hbm.at[pl.ds(off, _CHUNK)])
            pcp2.wait()
            pltpu.sync_copy(bufs2[pb], out2_hbm.at[pl.ds(off, _CHUNK)])
        pending = (c, b, cp1, cp2)
    pc, pb, pcp1, pcp2 = pending
    off = base + pc * _CHUNK
    pcp1.wait()
    pltpu.sync_copy(bufs1[pb], out1_hbm.at[pl.ds(off, _CHUNK)])
    pcp2.wait()
    pltpu.sync_copy(bufs2[pb], out2_hbm.at[pl.ds(off, _CHUNK)])


def kernel(idx, emb1, emb2):
    idx_r = idx.astype(jnp.int32).reshape(_NW, _NCHUNK, _CHUNK)
    out1, out2 = _lookup(idx_r, emb1, emb2)
    return (out1, out2)


# speedup vs baseline: 1.5581x; 1.0161x over previous
import functools

import jax
import jax.numpy as jnp
from jax import lax
from jax.experimental import pallas as pl
from jax.experimental.pallas import tpu as pltpu
from jax.experimental.pallas import tpu_sc as plsc

NNODES = 100000
DIM = 128
BATCH = 16384

_info = plsc.get_sparse_core_info()
_NC, _NS = _info.num_cores, _info.num_subcores
_NW = _NC * _NS
_B_PER_W = BATCH // _NW
_CHUNK = 128
_NCHUNK = _B_PER_W // _CHUNK
_NB = 3

_mesh = plsc.VectorSubcoreMesh(core_axis_name="c", subcore_axis_name="s")


@functools.partial(
    pl.kernel,
    mesh=_mesh,
    out_type=(jax.ShapeDtypeStruct((BATCH, DIM), jnp.float32),
              jax.ShapeDtypeStruct((BATCH, DIM), jnp.float32)),
    scratch_types=(
        [pltpu.VMEM((_NCHUNK, _CHUNK), jnp.int32)]
        + [pltpu.VMEM((_CHUNK, DIM), jnp.float32) for _ in range(2 * _NB)]
        + [pltpu.SemaphoreType.DMA for _ in range(4 * _NB)]
    ),
)
def _lookup(idx_hbm, emb1_hbm, emb2_hbm, out1_hbm, out2_hbm, idx_v, *rs):
    bufs1, bufs2 = rs[:_NB], rs[_NB:2 * _NB]
    sems = rs[2 * _NB:]
    g1s, g2s = sems[:_NB], sems[_NB:2 * _NB]
    w1s, w2s = sems[2 * _NB:3 * _NB], sems[3 * _NB:]

    wid = lax.axis_index("s") * _NC + lax.axis_index("c")
    base = wid * _B_PER_W
    pltpu.sync_copy(idx_hbm.at[wid], idx_v)

    gathers = [None] * _NCHUNK
    writes = [None] * _NCHUNK

    def issue_write(c):
        b = c % _NB
        p1, p2 = gathers[c]
        off = base + c * _CHUNK
        p1.wait()
        w1 = pltpu.async_copy(bufs1[b], out1_hbm.at[pl.ds(off, _CHUNK)], w1s[b])
        p2.wait()
        w2 = pltpu.async_copy(bufs2[b], out2_hbm.at[pl.ds(off, _CHUNK)], w2s[b])
        writes[c] = (w1, w2)

    for c in range(_NCHUNK):
        b = c % _NB
        if c >= _NB:
            pw1, pw2 = writes[c - _NB]
            pw1.wait()
            pw2.wait()
        cp1 = pltpu.async_copy(emb1_hbm.at[idx_v.at[c]], bufs1[b], g1s[b])
        cp2 = pltpu.async_copy(emb2_hbm.at[idx_v.at[c]], bufs2[b], g2s[b])
        gathers[c] = (cp1, cp2)
        if c >= 1:
            issue_write(c - 1)

    issue_write(_NCHUNK - 1)
    for c in range(max(0, _NCHUNK - _NB), _NCHUNK):
        pw1, pw2 = writes[c]
        pw1.wait()
        pw2.wait()


def kernel(idx, emb1, emb2):
    idx_r = idx.astype(jnp.int32).reshape(_NW, _NCHUNK, _CHUNK)
    out1, out2 = _lookup(idx_r, emb1, emb2)
    return (out1, out2)
